# baseline (speedup 1.0000x reference)
import functools

import jax
import jax.numpy as jnp
from jax import lax
from jax.experimental import pallas as pl
from jax.experimental.pallas import tpu as pltpu
from jax.experimental.pallas import tpu_sc as plsc

EMBED_DIM = 32
B = 16384
F = 26
VOCAB = 1_000_000
NC, NS = 2, 16
NW = NC * NS
CHUNK = B // NW
PACK = 128 // EMBED_DIM
PACKED_ROWS = VOCAB // PACK
N_BLOCKS = VOCAB // 128
MAIN_ITERS = N_BLOCKS // NW

_mesh = plsc.VectorSubcoreMesh(core_axis_name="c", subcore_axis_name="s")
_params = pltpu.CompilerParams(
    use_tc_tiling_on_sc=True,
    needs_layout_passes=False,
    disable_bounds_checks=True,
)


@functools.partial(
    pl.kernel,
    mesh=_mesh,
    out_type=jax.ShapeDtypeStruct((PACKED_ROWS, 128), jnp.float32),
    scratch_types=[
        pltpu.VMEM((2, 32, 128), jnp.float32),
        pltpu.VMEM((2, 32, 128), jnp.float32),
        pltpu.SemaphoreType.DMA((2,)),
        pltpu.SemaphoreType.DMA((2,)),
    ],
    compiler_params=_params,
)
def _repack_table(wt_hbm, wtail_hbm, out_hbm, in_v, tr_v, sems_i, sems_o):
    wid = lax.axis_index("s") * NC + lax.axis_index("c")
    iota = lax.iota(jnp.int32, 16)

    def in_copy(k_slot, blk, width=128):
        return pltpu.make_async_copy(
            wt_hbm.at[:, pl.ds(128 * blk, width)],
            in_v.at[k_slot, :, pl.ds(0, width)],
            sems_i.at[k_slot],
        )

    def out_copy(k_slot, blk, rows=32):
        return pltpu.make_async_copy(
            tr_v.at[k_slot, pl.ds(0, rows), :],
            out_hbm.at[pl.ds(32 * blk, rows), :],
            sems_o.at[k_slot],
        )

    row_lo = iota
    row_hi = iota + 16

    def transpose(k_slot):
        for q in range(32):
            for t in range(8):
                row_vec = row_lo if t % 2 == 0 else row_hi
                col_vec = jnp.full((16,), 4 * q + t // 2, jnp.int32)
                v = plsc.load_gather(in_v.at[k_slot], [row_vec, col_vec])
                tr_v[k_slot, q, pl.ds(16 * t, 16)] = v

    in_copy(0, wid).start()
    def main_body(k, _):
        blk = wid + NW * k
        slot = k % 2

        @pl.when(k < MAIN_ITERS - 1)
        def _():
            in_copy((k + 1) % 2, blk + NW).start()

        in_copy(slot, blk).wait()

        @pl.when(k >= 2)
        def _():
            out_copy(slot, blk).wait()

        transpose(slot)
        out_copy(slot, blk).start()
        return ()

    lax.fori_loop(0, MAIN_ITERS, main_body, ())
    out_copy(0, 0).wait()
    out_copy(1, 0).wait()

    @pl.when(wid < 4)
    def _():
        blk = N_BLOCKS - 4 + wid
        in_copy(0, blk).start()
        in_copy(0, blk).wait()
        transpose(0)
        out_copy(0, blk).start()
        out_copy(0, blk).wait()

    @pl.when(wid == 4)
    def _():
        def tail_in():
            return pltpu.make_async_copy(
                wtail_hbm, in_v.at[0, pl.ds(0, 16), :], sems_i.at[0]
            )

        def tail_out():
            return pltpu.make_async_copy(
                in_v.at[0, pl.ds(0, 16), :],
                out_hbm.at[pl.ds(PACKED_ROWS - 16, 16), :],
                sems_o.at[0],
            )

        tail_in().start()
        tail_in().wait()
        tail_out().start()
        tail_out().wait()


@functools.partial(
    pl.kernel,
    mesh=_mesh,
    out_type=jax.ShapeDtypeStruct((F, EMBED_DIM, B), jnp.float32),
    scratch_types=[
        pltpu.VMEM((2, CHUNK), jnp.int32),
        pltpu.VMEM((CHUNK,), jnp.int32),
        pltpu.VMEM((4, 128, 128), jnp.float32),
        pltpu.VMEM((2, 32, 128), jnp.float32),
        pltpu.SemaphoreType.DMA((2,)),
        [pltpu.SemaphoreType.DMA] * 4,
        [pltpu.SemaphoreType.DMA] * 2,
    ],
    compiler_params=_params,
)
def _embedding_gather(idxT_hbm, table_hbm, out_hbm, idx_v, j_v, rows_v,
                      stage_v, sems_i, sems_g, sems_o):
    wid = lax.axis_index("s") * NC + lax.axis_index("c")
    b0 = pl.multiple_of(wid * CHUNK, 128)
    iota = lax.iota(jnp.int32, 16)

    def idx_copy(f_slot, f):
        return pltpu.make_async_copy(
            idxT_hbm.at[f, pl.ds(b0, CHUNK)], idx_v.at[f_slot], sems_i.at[f_slot]
        )

    def gather_copy(blk, f_slot):
        return pltpu.make_async_copy(
            table_hbm.at[j_v.at[pl.ds(128 * blk, 128)]],
            rows_v.at[blk],
            sems_g[blk],
        )

    def out_copy(s_slot, f, blk):
        return pltpu.make_async_copy(
            stage_v.at[s_slot],
            out_hbm.at[f, :, pl.ds(b0 + 128 * blk, 128)],
            sems_o[s_slot],
        )

    idx_copy(0, 0).start()

    def f_body(f, _):
        f_slot = f % 2

        @pl.when(f < F - 1)
        def _():
            idx_copy((f + 1) % 2, f + 1).start()

        idx_copy(f_slot, f).wait()

        for r in range(CHUNK // 16):
            j_v[pl.ds(16 * r, 16)] = lax.shift_right_logical(
                idx_v[f_slot, pl.ds(16 * r, 16)], 2
            )

        for blk in range(4):
            gather_copy(blk, f_slot).start()

        for blk in range(4):
            gather_copy(blk, f_slot).wait()
            s_slot = blk % 2

            @pl.when((f * 4 + blk) >= 2)
            def _():
                out_copy(s_slot, f, blk).wait()

            for t in range(8):
                i_vec = idx_v[f_slot, pl.ds(128 * blk + 16 * t, 16)]
                col_lo = lax.shift_left(
                    lax.bitwise_and(i_vec, jnp.int32(3)), 5
                )
                row_vec = iota + 16 * t
                for e in range(EMBED_DIM):
                    v = plsc.load_gather(
                        rows_v.at[blk], [row_vec, col_lo + e]
                    )
                    stage_v[s_slot, e, pl.ds(16 * t, 16)] = v
            out_copy(s_slot, f, blk).start()
        return ()

    lax.fori_loop(0, F, f_body, ())
    out_copy(0, 0, 0).wait()
    out_copy(1, 0, 0).wait()


def kernel(input, weight):
    wtail = jnp.reshape(weight[128 * N_BLOCKS:], (16, 128))
    packed = _repack_table(weight.T, wtail)
    out = _embedding_gather(input.T, packed)
    return jnp.transpose(out, (2, 0, 1))


# speedup vs baseline: 1.8148x; 1.8148x over previous
import functools

import jax
import jax.numpy as jnp
from jax import lax
from jax.experimental import pallas as pl
from jax.experimental.pallas import tpu as pltpu
from jax.experimental.pallas import tpu_sc as plsc

EMBED_DIM = 32
B = 16384
F = 26
VOCAB = 1_000_000
NC, NS = 2, 16
NW = NC * NS
CHUNK = B // NW
PACK = 128 // EMBED_DIM
PACKED_ROWS = VOCAB // PACK
N_BLOCKS = VOCAB // 128
MAIN_ITERS = N_BLOCKS // NW

_mesh = plsc.VectorSubcoreMesh(core_axis_name="c", subcore_axis_name="s")
_params = pltpu.CompilerParams(
    use_tc_tiling_on_sc=True,
    needs_layout_passes=False,
    disable_bounds_checks=True,
)


@functools.partial(
    pl.kernel,
    mesh=_mesh,
    out_type=jax.ShapeDtypeStruct((PACKED_ROWS, 128), jnp.float32),
    scratch_types=[
        pltpu.VMEM((2, 32, 128), jnp.float32),
        pltpu.VMEM((2, 32, 128), jnp.float32),
        pltpu.SemaphoreType.DMA((2,)),
        pltpu.SemaphoreType.DMA((2,)),
    ],
    compiler_params=_params,
)
def _repack_table(wt_hbm, wtail_hbm, out_hbm, in_v, tr_v, sems_i, sems_o):
    wid = lax.axis_index("s") * NC + lax.axis_index("c")
    iota = lax.iota(jnp.int32, 16)

    def in_copy(k_slot, blk, width=128):
        return pltpu.make_async_copy(
            wt_hbm.at[:, pl.ds(128 * blk, width)],
            in_v.at[k_slot, :, pl.ds(0, width)],
            sems_i.at[k_slot],
        )

    def out_copy(k_slot, blk, rows=32):
        return pltpu.make_async_copy(
            tr_v.at[k_slot, pl.ds(0, rows), :],
            out_hbm.at[pl.ds(32 * blk, rows), :],
            sems_o.at[k_slot],
        )

    row_lo = iota
    row_hi = iota + 16
    zeros = jnp.full((16,), 0, jnp.int32)

    def transpose(k_slot):
        @plsc.parallel_loop(0, 32, unroll=4)
        def _q_loop(q):
            for t in range(8):
                row_vec = row_lo if t % 2 == 0 else row_hi
                col_vec = zeros + (4 * q + t // 2)
                v = plsc.load_gather(in_v.at[k_slot], [row_vec, col_vec])
                tr_v[k_slot, q, pl.ds(16 * t, 16)] = v

    in_copy(0, wid).start()
    def main_body(k, _):
        blk = wid + NW * k
        slot = k % 2

        @pl.when(k < MAIN_ITERS - 1)
        def _():
            in_copy((k + 1) % 2, blk + NW).start()

        in_copy(slot, blk).wait()

        @pl.when(k >= 2)
        def _():
            out_copy(slot, blk).wait()

        transpose(slot)
        out_copy(slot, blk).start()
        return ()

    lax.fori_loop(0, MAIN_ITERS, main_body, ())
    out_copy(0, 0).wait()
    out_copy(1, 0).wait()

    @pl.when(wid < 4)
    def _():
        blk = N_BLOCKS - 4 + wid
        in_copy(0, blk).start()
        in_copy(0, blk).wait()
        transpose(0)
        out_copy(0, blk).start()
        out_copy(0, blk).wait()

    @pl.when(wid == 4)
    def _():
        def tail_in():
            return pltpu.make_async_copy(
                wtail_hbm, in_v.at[0, pl.ds(0, 16), :], sems_i.at[0]
            )

        def tail_out():
            return pltpu.make_async_copy(
                in_v.at[0, pl.ds(0, 16), :],
                out_hbm.at[pl.ds(PACKED_ROWS - 16, 16), :],
                sems_o.at[0],
            )

        tail_in().start()
        tail_in().wait()
        tail_out().start()
        tail_out().wait()


@functools.partial(
    pl.kernel,
    mesh=_mesh,
    out_type=jax.ShapeDtypeStruct((F, EMBED_DIM, B), jnp.float32),
    scratch_types=[
        pltpu.VMEM((2, CHUNK), jnp.int32),
        pltpu.VMEM((CHUNK,), jnp.int32),
        pltpu.VMEM((4, 128, 128), jnp.float32),
        pltpu.VMEM((2, 32, 128), jnp.float32),
        pltpu.SemaphoreType.DMA((2,)),
        [pltpu.SemaphoreType.DMA] * 4,
        [pltpu.SemaphoreType.DMA] * 2,
    ],
    compiler_params=_params,
)
def _embedding_gather(idxT_hbm, table_hbm, out_hbm, idx_v, j_v, rows_v,
                      stage_v, sems_i, sems_g, sems_o):
    wid = lax.axis_index("s") * NC + lax.axis_index("c")
    b0 = pl.multiple_of(wid * CHUNK, 128)
    iota = lax.iota(jnp.int32, 16)

    def idx_copy(f_slot, f):
        return pltpu.make_async_copy(
            idxT_hbm.at[f, pl.ds(b0, CHUNK)], idx_v.at[f_slot], sems_i.at[f_slot]
        )

    def gather_copy(blk, f_slot):
        return pltpu.make_async_copy(
            table_hbm.at[j_v.at[pl.ds(128 * blk, 128)]],
            rows_v.at[blk],
            sems_g[blk],
        )

    def out_copy(s_slot, f, blk):
        return pltpu.make_async_copy(
            stage_v.at[s_slot],
            out_hbm.at[f, :, pl.ds(b0 + 128 * blk, 128)],
            sems_o[s_slot],
        )

    idx_copy(0, 0).start()

    def f_body(f, _):
        f_slot = f % 2

        @pl.when(f < F - 1)
        def _():
            idx_copy((f + 1) % 2, f + 1).start()

        idx_copy(f_slot, f).wait()

        @plsc.parallel_loop(0, CHUNK // 16, unroll=4)
        def _j_loop(r):
            j_v[pl.ds(16 * r, 16)] = lax.shift_right_logical(
                idx_v[f_slot, pl.ds(16 * r, 16)], 2
            )

        for blk in range(4):
            gather_copy(blk, f_slot).start()

        for blk in range(4):
            gather_copy(blk, f_slot).wait()
            s_slot = blk % 2

            @pl.when((f * 4 + blk) >= 2)
            def _():
                out_copy(s_slot, f, blk).wait()

            @plsc.parallel_loop(0, 8, unroll=2)
            def _t_loop(t):
                i_vec = idx_v[f_slot, pl.ds(128 * blk + 16 * t, 16)]
                col_lo = lax.shift_left(
                    lax.bitwise_and(i_vec, jnp.int32(3)), 5
                )
                row_vec = iota + 16 * t
                for e in range(EMBED_DIM):
                    v = plsc.load_gather(
                        rows_v.at[blk], [row_vec, col_lo + e]
                    )
                    stage_v[s_slot, e, pl.ds(16 * t, 16)] = v
            out_copy(s_slot, f, blk).start()
        return ()

    lax.fori_loop(0, F, f_body, ())
    out_copy(0, 0, 0).wait()
    out_copy(1, 0, 0).wait()


def kernel(input, weight):
    wtail = jnp.reshape(weight[128 * N_BLOCKS:], (16, 128))
    packed = _repack_table(weight.T, wtail)
    out = _embedding_gather(input.T, packed)
    return jnp.transpose(out, (2, 0, 1))
